# baseline (speedup 1.0000x reference)
import jax
import jax.numpy as jnp
from jax.experimental import pallas as pl
from jax.experimental.pallas import tpu as pltpu

SEQ = 8192
DIM = 1024
_NDMA = 8
_ROWS = SEQ // _NDMA


def _dma_body(src, dst, sems):
    copies = [
        pltpu.make_async_copy(
            src.at[pl.ds(i * _ROWS, _ROWS)],
            dst.at[pl.ds(i * _ROWS, _ROWS)],
            sems.at[i],
        )
        for i in range(_NDMA)
    ]
    for c in copies:
        c.start()
    for c in copies:
        c.wait()


def kernel(input, embed):
    out = pl.pallas_call(
        _dma_body,
        out_shape=jax.ShapeDtypeStruct((SEQ, DIM), embed.dtype),
        in_specs=[pl.BlockSpec(memory_space=pl.ANY)],
        out_specs=pl.BlockSpec(memory_space=pl.ANY),
        scratch_shapes=[pltpu.SemaphoreType.DMA((_NDMA,))],
    )(embed)
    return out.reshape(SEQ, 1, DIM)


# speedup vs baseline: 14.4482x; 14.4482x over previous
import functools

import jax
import jax.numpy as jnp
from jax import lax
from jax.experimental import pallas as pl
from jax.experimental.pallas import tpu as pltpu
from jax.experimental.pallas import tpu_sc as plsc

SEQ = 8192
DIM = 1024

_NUM_CORES = 2
_ROWS_PER_C = SEQ // _NUM_CORES
_CHUNK = 512
_NBUF = 3
_NCHUNK = _ROWS_PER_C // _CHUNK

_mesh = plsc.ScalarSubcoreMesh(axis_name="c", num_cores=_NUM_CORES)


@functools.partial(
    pl.kernel,
    mesh=_mesh,
    out_type=jax.ShapeDtypeStruct((SEQ, DIM), jnp.float32),
    scratch_types=(
        [pltpu.VMEM_SHARED((_CHUNK, DIM), jnp.float32) for _ in range(_NBUF)]
        + [pltpu.SemaphoreType.DMA for _ in range(2 * _NBUF)]
    ),
)
def _sc_copy(embed_hbm, out_hbm, *scratch):
    bufs = scratch[:_NBUF]
    isems = scratch[_NBUF:2 * _NBUF]
    osems = scratch[2 * _NBUF:]
    base = lax.axis_index("c") * _ROWS_PER_C

    def in_copy(i):
        return pltpu.async_copy(
            embed_hbm.at[pl.ds(base + i * _CHUNK, _CHUNK)],
            bufs[i % _NBUF],
            isems[i % _NBUF],
        )

    def out_copy(i):
        return pltpu.async_copy(
            bufs[i % _NBUF],
            out_hbm.at[pl.ds(base + i * _CHUNK, _CHUNK)],
            osems[i % _NBUF],
        )

    ins = [None] * _NCHUNK
    outs = [None] * _NCHUNK
    for i in range(min(_NBUF, _NCHUNK)):
        ins[i] = in_copy(i)
    for i in range(_NCHUNK):
        ins[i].wait()
        outs[i] = out_copy(i)
        nxt = i + _NBUF
        if nxt < _NCHUNK:
            outs[i].wait()
            ins[nxt] = in_copy(nxt)
    for i in range(max(0, _NCHUNK - _NBUF), _NCHUNK):
        outs[i].wait()


def kernel(input, embed):
    return _sc_copy(embed).reshape(SEQ, 1, DIM)
